# baseline (speedup 1.0000x reference)
import jax
import jax.numpy as jnp
from jax import lax
from jax.experimental import pallas as pl
from jax.experimental.pallas import tpu as pltpu

_EPS = 1e-5
_RATES = (6, 12, 18)
_H = 64
_HW = _H * _H
_PAD = 1280
_XPW = _HW + 2 * _PAD


def _asppup_kernel(x_ref, wtb_ref, w0_ref, bb_ref, wp_ref, bp_ref, o_ref,
                   xs_ref, stk_ref, sems):
    cin = x_ref.shape[1]
    xs_ref[:, :_PAD] = jnp.zeros((cin, _PAD), jnp.bfloat16)
    xs_ref[:, _PAD + _HW:] = jnp.zeros((cin, _PAD), jnp.bfloat16)
    xs_ref[:, _PAD:_PAD + _HW] = x_ref[0].astype(jnp.bfloat16)

    copies = []
    for bi, d in enumerate(_RATES):
        for irh, rh in enumerate((-d, 0, d)):
            cp = pltpu.make_async_copy(
                xs_ref.at[:, _PAD + rh * _H:_PAD + rh * _H + _HW],
                stk_ref.at[768 * bi + 256 * irh:768 * bi + 256 * (irh + 1), :],
                sems.at[3 * bi + irh])
            cp.start()
            copies.append(cp)

    col = lax.broadcasted_iota(jnp.int32, (1, _HW), 1) % _H
    wp = wp_ref[...]

    def tail(br, acc):
        a = jnp.maximum(acc + bb_ref[br, :, 0:1], 0.0)
        z = jnp.dot(wp, a.astype(jnp.bfloat16),
                    preferred_element_type=jnp.float32)
        o_ref[0, br] = jnp.maximum(z + bp_ref[:, 0:1], 0.0).astype(jnp.bfloat16)

    copies[1].wait()
    tail(0, jnp.dot(w0_ref[...], stk_ref[256:512, :],
                    preferred_element_type=jnp.float32))
    for bi, d in enumerate(_RATES):
        for irh in range(3):
            if not (bi == 0 and irh == 1):
                copies[3 * bi + irh].wait()
        y = jnp.dot(wtb_ref[384 * bi:384 * (bi + 1), :],
                    stk_ref[768 * bi:768 * (bi + 1), :],
                    preferred_element_type=jnp.float32)
        acc = None
        for icw, cw in enumerate((-d, 0, d)):
            u = y[128 * icw:128 * (icw + 1)]
            if cw == 0:
                g = u
            else:
                rolled = jnp.concatenate([u[:, cw:], u[:, :cw]], axis=1)
                if cw > 0:
                    g = jnp.where(col < _H - cw, rolled, 0.0)
                else:
                    g = jnp.where(col >= -cw, rolled, 0.0)
            acc = g if acc is None else acc + g
        tail(bi + 1, acc)


def kernel(x, w0, w1, w2, w3, wp,
           g0, b0, m0, v0, g1, b1, m1, v1,
           g2, b2, m2, v2, g3, b3, m3, v3,
           gp, bp, mp, vp):
    B, Cin, H, W = x.shape
    Cout = w0.shape[0]

    x2 = x.reshape(B, Cin, H * W)

    def fold(w, g, b, m, v):
        s = g * lax.rsqrt(v + _EPS)
        return w * s[:, None, None, None], b - m * s

    w0f, bias0 = fold(w0, g0, b0, m0, v0)
    blocks = []
    biases = [bias0]
    for w, g, b, m, v in ((w1, g1, b1, m1, v1),
                          (w2, g2, b2, m2, v2),
                          (w3, g3, b3, m3, v3)):
        wf, bi = fold(w, g, b, m, v)
        blocks.append(jnp.concatenate(
            [jnp.concatenate([wf[:, :, kh, kw] for kh in range(3)], axis=1)
             for kw in range(3)], axis=0))
        biases.append(bi)
    wtb = jnp.concatenate(blocks, axis=0).astype(jnp.bfloat16)
    w0b = w0f[:, :, 0, 0].astype(jnp.bfloat16)
    bb = jnp.broadcast_to(jnp.stack(biases)[:, :, None], (4, Cout, 128))
    sp = gp * lax.rsqrt(vp + _EPS)
    wpf = (wp[:, :, 0, 0] * sp[:, None]).astype(jnp.bfloat16)
    bpf = jnp.broadcast_to((bp - mp * sp)[:, None], (Cout, 128))

    out = pl.pallas_call(
        _asppup_kernel,
        grid=(B,),
        in_specs=[
            pl.BlockSpec((1, Cin, _HW), lambda b: (b, 0, 0)),
            pl.BlockSpec((9 * Cout, 3 * Cin), lambda b: (0, 0)),
            pl.BlockSpec((Cout, Cin), lambda b: (0, 0)),
            pl.BlockSpec((4, Cout, 128), lambda b: (0, 0, 0)),
            pl.BlockSpec((Cout, Cout), lambda b: (0, 0)),
            pl.BlockSpec((Cout, 128), lambda b: (0, 0)),
        ],
        out_specs=pl.BlockSpec((1, 4, Cout, _HW), lambda b: (b, 0, 0, 0)),
        out_shape=jax.ShapeDtypeStruct((B, 4, Cout, _HW), jnp.bfloat16),
        scratch_shapes=[pltpu.VMEM((Cin, _XPW), jnp.bfloat16),
                        pltpu.VMEM((9 * Cin, _HW), jnp.bfloat16),
                        pltpu.SemaphoreType.DMA((9,))],
        compiler_params=pltpu.CompilerParams(
            dimension_semantics=("parallel",),
            vmem_limit_bytes=52 * 1024 * 1024,
        ),
    )(x2, wtb, w0b, bb, wpf, bpf)

    z = out.reshape(B, 2, 2, Cout, H, W).transpose(0, 3, 4, 1, 5, 2)
    return z.reshape(B, Cout, 2 * H, 2 * W).astype(jnp.float32)


# speedup vs baseline: 1.0306x; 1.0306x over previous
import jax
import jax.numpy as jnp
from jax import lax
from jax.experimental import pallas as pl
from jax.experimental.pallas import tpu as pltpu

_EPS = 1e-5
_RATES = (6, 12, 18)
_H = 64
_HW = _H * _H
_PAD = 1280
_XPW = _HW + 2 * _PAD


def _asppup_kernel(x_ref, wtb_ref, w0_ref, bb_ref, wp_ref, bp_ref, o_ref,
                   xs_ref, stk_ref):
    cin = x_ref.shape[1]
    xs_ref[:, :_PAD] = jnp.zeros((cin, _PAD), jnp.bfloat16)
    xs_ref[:, _PAD + _HW:] = jnp.zeros((cin, _PAD), jnp.bfloat16)
    xs_ref[:, _PAD:_PAD + _HW] = x_ref[0]

    for bi, d in enumerate(_RATES):
        for irh, rh in enumerate((-d, 0, d)):
            stk_ref[768 * bi + 256 * irh:768 * bi + 256 * (irh + 1), :] = (
                xs_ref[:, _PAD + rh * _H:_PAD + rh * _H + _HW])

    col = lax.broadcasted_iota(jnp.int32, (1, _HW), 1) % _H
    wp = wp_ref[...]

    def tail(br, acc):
        a = jnp.maximum(acc + bb_ref[br, :, 0:1], 0.0)
        z = jnp.dot(wp, a.astype(jnp.bfloat16),
                    preferred_element_type=jnp.float32)
        o_ref[0, br] = jnp.maximum(z + bp_ref[:, 0:1], 0.0).astype(jnp.bfloat16)

    tail(0, jnp.dot(w0_ref[...], stk_ref[256:512, :],
                    preferred_element_type=jnp.float32))
    for bi, d in enumerate(_RATES):
        y = jnp.dot(wtb_ref[384 * bi:384 * (bi + 1), :],
                    stk_ref[768 * bi:768 * (bi + 1), :],
                    preferred_element_type=jnp.float32)
        acc = None
        for icw, cw in enumerate((-d, 0, d)):
            u = y[128 * icw:128 * (icw + 1)]
            if cw == 0:
                g = u
            else:
                rolled = jnp.concatenate([u[:, cw:], u[:, :cw]], axis=1)
                if cw > 0:
                    g = jnp.where(col < _H - cw, rolled, 0.0)
                else:
                    g = jnp.where(col >= -cw, rolled, 0.0)
            acc = g if acc is None else acc + g
        tail(bi + 1, acc)


def kernel(x, w0, w1, w2, w3, wp,
           g0, b0, m0, v0, g1, b1, m1, v1,
           g2, b2, m2, v2, g3, b3, m3, v3,
           gp, bp, mp, vp):
    B, Cin, H, W = x.shape
    Cout = w0.shape[0]

    x2 = x.reshape(B, Cin, H * W).astype(jnp.bfloat16)

    def fold(w, g, b, m, v):
        s = g * lax.rsqrt(v + _EPS)
        return w * s[:, None, None, None], b - m * s

    w0f, bias0 = fold(w0, g0, b0, m0, v0)
    blocks = []
    biases = [bias0]
    for w, g, b, m, v in ((w1, g1, b1, m1, v1),
                          (w2, g2, b2, m2, v2),
                          (w3, g3, b3, m3, v3)):
        wf, bi = fold(w, g, b, m, v)
        blocks.append(jnp.concatenate(
            [jnp.concatenate([wf[:, :, kh, kw] for kh in range(3)], axis=1)
             for kw in range(3)], axis=0))
        biases.append(bi)
    wtb = jnp.concatenate(blocks, axis=0).astype(jnp.bfloat16)
    w0b = w0f[:, :, 0, 0].astype(jnp.bfloat16)
    bb = jnp.broadcast_to(jnp.stack(biases)[:, :, None], (4, Cout, 128))
    sp = gp * lax.rsqrt(vp + _EPS)
    wpf = (wp[:, :, 0, 0] * sp[:, None]).astype(jnp.bfloat16)
    bpf = jnp.broadcast_to((bp - mp * sp)[:, None], (Cout, 128))

    out = pl.pallas_call(
        _asppup_kernel,
        grid=(B,),
        in_specs=[
            pl.BlockSpec((1, Cin, _HW), lambda b: (b, 0, 0)),
            pl.BlockSpec((9 * Cout, 3 * Cin), lambda b: (0, 0)),
            pl.BlockSpec((Cout, Cin), lambda b: (0, 0)),
            pl.BlockSpec((4, Cout, 128), lambda b: (0, 0, 0)),
            pl.BlockSpec((Cout, Cout), lambda b: (0, 0)),
            pl.BlockSpec((Cout, 128), lambda b: (0, 0)),
        ],
        out_specs=pl.BlockSpec((1, 4, Cout, _HW), lambda b: (b, 0, 0, 0)),
        out_shape=jax.ShapeDtypeStruct((B, 4, Cout, _HW), jnp.bfloat16),
        scratch_shapes=[pltpu.VMEM((Cin, _XPW), jnp.bfloat16),
                        pltpu.VMEM((9 * Cin, _HW), jnp.bfloat16)],
        compiler_params=pltpu.CompilerParams(
            dimension_semantics=("parallel",),
            vmem_limit_bytes=52 * 1024 * 1024,
        ),
    )(x2, wtb, w0b, bb, wpf, bpf)

    z = out.reshape(B, 2, 2, Cout, H, W).transpose(0, 3, 4, 1, 5, 2)
    return z.reshape(B, Cout, 2 * H, 2 * W).astype(jnp.float32)


# speedup vs baseline: 1.0591x; 1.0276x over previous
import jax
import jax.numpy as jnp
from jax import lax
from jax.experimental import pallas as pl
from jax.experimental.pallas import tpu as pltpu

_EPS = 1e-5
_RATES = (6, 12, 18)
_H = 64
_HW = _H * _H
_PAD = 1280
_XPW = _HW + 2 * _PAD


def _asppup_kernel(x_ref, wtb_ref, w0_ref, bb_ref, wp_ref, bp_ref, o_ref,
                   xs_ref, stk_ref):
    cin = x_ref.shape[1]
    xs_ref[:, :_PAD] = jnp.zeros((cin, _PAD), jnp.bfloat16)
    xs_ref[:, _PAD + _HW:] = jnp.zeros((cin, _PAD), jnp.bfloat16)
    xs_ref[:, _PAD:_PAD + _HW] = x_ref[0].astype(jnp.bfloat16)

    for bi, d in enumerate(_RATES):
        for irh, rh in enumerate((-d, 0, d)):
            stk_ref[768 * bi + 256 * irh:768 * bi + 256 * (irh + 1), :] = (
                xs_ref[:, _PAD + rh * _H:_PAD + rh * _H + _HW])

    col = lax.broadcasted_iota(jnp.int32, (1, _HW), 1) % _H
    masks = {}
    for d in _RATES:
        masks[d] = jnp.where(col < _H - d, 1.0, 0.0)
        masks[-d] = jnp.where(col >= d, 1.0, 0.0)
    wp = wp_ref[...]

    def tail(br, acc):
        a = jnp.maximum(acc + bb_ref[br, :, 0:1], 0.0)
        z = jnp.dot(wp, a.astype(jnp.bfloat16),
                    preferred_element_type=jnp.float32)
        o_ref[0, br] = jnp.maximum(z + bp_ref[:, 0:1], 0.0).astype(jnp.bfloat16)

    tail(0, jnp.dot(w0_ref[...], stk_ref[256:512, :],
                    preferred_element_type=jnp.float32))
    for bi, d in enumerate(_RATES):
        y = jnp.dot(wtb_ref[384 * bi:384 * (bi + 1), :],
                    stk_ref[768 * bi:768 * (bi + 1), :],
                    preferred_element_type=jnp.float32)
        acc = None
        for icw, cw in enumerate((-d, 0, d)):
            u = y[128 * icw:128 * (icw + 1)]
            if cw == 0:
                g = u
            else:
                rolled = jnp.concatenate([u[:, cw:], u[:, :cw]], axis=1)
                g = rolled * masks[cw]
            acc = g if acc is None else acc + g
        tail(bi + 1, acc)


def kernel(x, w0, w1, w2, w3, wp,
           g0, b0, m0, v0, g1, b1, m1, v1,
           g2, b2, m2, v2, g3, b3, m3, v3,
           gp, bp, mp, vp):
    B, Cin, H, W = x.shape
    Cout = w0.shape[0]

    x2 = x.reshape(B, Cin, H * W)

    def fold(w, g, b, m, v):
        s = g * lax.rsqrt(v + _EPS)
        return w * s[:, None, None, None], b - m * s

    w0f, bias0 = fold(w0, g0, b0, m0, v0)
    blocks = []
    biases = [bias0]
    for w, g, b, m, v in ((w1, g1, b1, m1, v1),
                          (w2, g2, b2, m2, v2),
                          (w3, g3, b3, m3, v3)):
        wf, bi = fold(w, g, b, m, v)
        blocks.append(jnp.concatenate(
            [jnp.concatenate([wf[:, :, kh, kw] for kh in range(3)], axis=1)
             for kw in range(3)], axis=0))
        biases.append(bi)
    wtb = jnp.concatenate(blocks, axis=0).astype(jnp.bfloat16)
    w0b = w0f[:, :, 0, 0].astype(jnp.bfloat16)
    bb = jnp.broadcast_to(jnp.stack(biases)[:, :, None], (4, Cout, 128))
    sp = gp * lax.rsqrt(vp + _EPS)
    wpf = (wp[:, :, 0, 0] * sp[:, None]).astype(jnp.bfloat16)
    bpf = jnp.broadcast_to((bp - mp * sp)[:, None], (Cout, 128))

    out = pl.pallas_call(
        _asppup_kernel,
        grid=(B,),
        in_specs=[
            pl.BlockSpec((1, Cin, _HW), lambda b: (b, 0, 0)),
            pl.BlockSpec((9 * Cout, 3 * Cin), lambda b: (0, 0)),
            pl.BlockSpec((Cout, Cin), lambda b: (0, 0)),
            pl.BlockSpec((4, Cout, 128), lambda b: (0, 0, 0)),
            pl.BlockSpec((Cout, Cout), lambda b: (0, 0)),
            pl.BlockSpec((Cout, 128), lambda b: (0, 0)),
        ],
        out_specs=pl.BlockSpec((1, 4, Cout, _HW), lambda b: (b, 0, 0, 0)),
        out_shape=jax.ShapeDtypeStruct((B, 4, Cout, _HW), jnp.bfloat16),
        scratch_shapes=[pltpu.VMEM((Cin, _XPW), jnp.bfloat16),
                        pltpu.VMEM((9 * Cin, _HW), jnp.bfloat16)],
        compiler_params=pltpu.CompilerParams(
            dimension_semantics=("parallel",),
            vmem_limit_bytes=52 * 1024 * 1024,
        ),
    )(x2, wtb, w0b, bb, wpf, bpf)

    z = out.reshape(B, 2, 2, Cout, H, W).transpose(0, 3, 4, 1, 5, 2)
    return z.reshape(B, Cout, 2 * H, 2 * W).astype(jnp.float32)
